# Initial kernel scaffold; baseline (speedup 1.0000x reference)
import jax
import jax.numpy as jnp
from jax.experimental import pallas as pl


def kernel(boxes, scores):
    raise NotImplementedError("write your pallas kernel here")


# speedup vs baseline: 18.1826x; 18.1826x over previous
import functools

import jax
import jax.numpy as jnp
from jax.experimental import pallas as pl
from jax.experimental.pallas import tpu as pltpu

CONF_THRES = 0.25
IOU_THRES = 0.45
MAX_DET = 300

N = 20000
NPAD = 20480
R, C = 160, 128
OUT_ROWS = 304


def _nms_body(x1_ref, y1_ref, x2_ref, y2_ref, s_ref, out_ref,
              work_ref, area2_ref, lin_ref):
    s = s_ref[...]
    work_ref[...] = jnp.where(s >= CONF_THRES, s, -jnp.inf)
    x1 = x1_ref[...]
    y1 = y1_ref[...]
    x2 = x2_ref[...]
    y2 = y2_ref[...]
    area2_ref[...] = (x2 - x1) * (y2 - y1)
    row = jax.lax.broadcasted_iota(jnp.int32, (R, C), 0)
    col = jax.lax.broadcasted_iota(jnp.int32, (R, C), 1)
    lin_ref[...] = row * C + col

    lane = jax.lax.broadcasted_iota(jnp.int32, (1, C), 1)

    def body(i, _):
        work = work_ref[...]
        lin = lin_ref[...]
        m = jnp.max(work)
        elig = work == m
        j = jnp.min(jnp.where(elig, lin, jnp.int32(2**30)))
        sel = lin == j
        selm = sel & elig
        x1j = jnp.sum(jnp.where(selm, x1_ref[...], 0.0))
        y1j = jnp.sum(jnp.where(selm, y1_ref[...], 0.0))
        x2j = jnp.sum(jnp.where(selm, x2_ref[...], 0.0))
        y2j = jnp.sum(jnp.where(selm, y2_ref[...], 0.0))

        ltx = jnp.maximum(x1j, x1_ref[...])
        lty = jnp.maximum(y1j, y1_ref[...])
        rbx = jnp.minimum(x2j, x2_ref[...])
        rby = jnp.minimum(y2j, y2_ref[...])
        w = jnp.clip(rbx - ltx, 0.0, None)
        h = jnp.clip(rby - lty, 0.0, None)
        inter = w * h
        area1 = (x2j - x1j) * (y2j - y1j)
        iou = inter / (area1 + area2_ref[...] - inter + 1e-9)
        work_ref[...] = jnp.where((iou > IOU_THRES) | sel, -jnp.inf, work)

        finite = m > -jnp.inf
        vals = jnp.where(lane == 0, x1j,
               jnp.where(lane == 1, y1j,
               jnp.where(lane == 2, x2j,
               jnp.where(lane == 3, y2j, m))))
        out_ref[pl.ds(i, 1), :] = jnp.where(finite, vals, 0.0)
        return 0

    jax.lax.fori_loop(0, MAX_DET, body, 0)


@jax.jit
def kernel(boxes, scores):
    bp = jnp.pad(boxes, ((0, NPAD - N), (0, 0)))
    planes = bp.T.reshape(4, R, C)
    sp = jnp.pad(scores, (0, NPAD - N)).reshape(R, C)
    out = pl.pallas_call(
        _nms_body,
        out_shape=jax.ShapeDtypeStruct((OUT_ROWS, C), jnp.float32),
        scratch_shapes=[
            pltpu.VMEM((R, C), jnp.float32),
            pltpu.VMEM((R, C), jnp.float32),
            pltpu.VMEM((R, C), jnp.int32),
        ],
    )(planes[0], planes[1], planes[2], planes[3], sp)
    return out[:MAX_DET, :5]
